# Initial kernel scaffold; baseline (speedup 1.0000x reference)
import functools

import jax
import jax.numpy as jnp
from jax import lax
from jax.experimental import pallas as pl
from jax.experimental.pallas import tpu as pltpu
from jax.experimental.pallas import tpu_sc as plsc

_G = 64
_R = 256
_TILES = 16
_CW = 128
_K = 8


def _ceil_to(a, b):
    return (a + b - 1) // b * b


def _sc_seg(NP, CPT, Hh):
    RPT = NP // _TILES
    mesh = plsc.VectorSubcoreMesh(core_axis_name="c", subcore_axis_name="s")

    @functools.partial(
        pl.kernel,
        out_type=[jax.ShapeDtypeStruct((NP, Hh), jnp.float32)] * 2,
        mesh=mesh,
        scratch_types=[
            pltpu.VMEM((_K, _CW), jnp.int32),
            pltpu.VMEM((_K, _CW), jnp.int32),
            pltpu.VMEM((_K, _CW, Hh), jnp.float32),
            pltpu.VMEM_SHARED((NP, Hh), jnp.float32),
            pltpu.SemaphoreType.DMA,
        ],
    )
    def seg(xa, xb, srcr, dstr, z, sa, sb, sbuf, dbuf, rbuf, acc, sem):
        c = lax.axis_index("c")
        s = lax.axis_index("s")
        pltpu.sync_copy(z.at[pl.ds(s * RPT, RPT)], acc.at[pl.ds(s * RPT, RPT)])
        plsc.subcore_barrier()

        def run(table, out):
            def body(g, carry):
                row0 = s * CPT + g * _K
                pltpu.sync_copy(srcr.at[pl.ds(row0, _K)], sbuf)
                pltpu.sync_copy(dstr.at[pl.ds(row0, _K)], dbuf)
                cps = [
                    pltpu.async_copy(table.at[sbuf.at[j]], rbuf.at[j], sem)
                    for j in range(_K)
                ]
                for cp in cps:
                    cp.wait()
                for j in range(_K):
                    pltpu.sync_copy(rbuf.at[j], acc.at[dbuf.at[j]], add=True)
                return carry

            lax.fori_loop(0, CPT // _K, body, 0)
            plsc.subcore_barrier()
            pltpu.sync_copy(acc.at[pl.ds(s * RPT, RPT)],
                            out.at[pl.ds(s * RPT, RPT)])

        @pl.when(c == 0)
        def _():
            run(xa, sa)

        @pl.when(c == 1)
        def _():
            run(xb, sb)

    return seg


def _sc_degt(NP, CPT, EF):
    RPT = NP // _TILES
    mesh = plsc.VectorSubcoreMesh(core_axis_name="c", subcore_axis_name="s")

    @functools.partial(
        pl.kernel,
        out_type=[jax.ShapeDtypeStruct((NP, EF), jnp.float32)] * 2,
        mesh=mesh,
        scratch_types=[
            pltpu.VMEM((_K, _CW), jnp.int32),
            pltpu.VMEM((_K, _CW, EF), jnp.float32),
            pltpu.VMEM((_CW, EF), jnp.float32),
            pltpu.VMEM_SHARED((NP, EF), jnp.float32),
        ],
    )
    def degt(ear, dstr, z, ones, t_out, deg_out, dbuf, eabuf, obuf, acc):
        c = lax.axis_index("c")
        s = lax.axis_index("s")
        pltpu.sync_copy(z.at[pl.ds(s * RPT, RPT)], acc.at[pl.ds(s * RPT, RPT)])

        @pl.when(c == 1)
        def _():
            pltpu.sync_copy(ones, obuf)

        plsc.subcore_barrier()

        def run(use_ea):
            def body(g, carry):
                row0 = s * CPT + g * _K
                pltpu.sync_copy(dstr.at[pl.ds(row0, _K)], dbuf)
                if use_ea:
                    pltpu.sync_copy(ear.at[pl.ds(row0, _K)], eabuf)
                for j in range(_K):
                    src = eabuf.at[j] if use_ea else obuf
                    pltpu.sync_copy(src, acc.at[dbuf.at[j]], add=True)
                return carry

            lax.fori_loop(0, CPT // _K, body, 0)
            plsc.subcore_barrier()

        @pl.when(c == 0)
        def _():
            run(True)
            pltpu.sync_copy(acc.at[pl.ds(s * RPT, RPT)],
                            t_out.at[pl.ds(s * RPT, RPT)])

        @pl.when(c == 1)
        def _():
            run(False)
            pltpu.sync_copy(acc.at[pl.ds(s * RPT, RPT)],
                            deg_out.at[pl.ds(s * RPT, RPT)])

    return degt


def _dense_in(x_p, Wn, bn):
    NP, F = x_p.shape
    H = Wn.shape[1]
    Hh = H // 2

    def body(x_ref, w_ref, b_ref, ya_ref, yb_ref):
        y = jnp.dot(x_ref[...], w_ref[...],
                    preferred_element_type=jnp.float32) + b_ref[...]
        ya_ref[...] = y[:, :Hh]
        yb_ref[...] = y[:, Hh:]

    return pl.pallas_call(
        body,
        grid=(NP // _R,),
        in_specs=[
            pl.BlockSpec((_R, F), lambda i: (i, 0)),
            pl.BlockSpec((F, H), lambda i: (0, 0)),
            pl.BlockSpec((1, H), lambda i: (0, 0)),
        ],
        out_specs=[pl.BlockSpec((_R, Hh), lambda i: (i, 0))] * 2,
        out_shape=[jax.ShapeDtypeStruct((NP, Hh), jnp.float32)] * 2,
    )(x_p, Wn, bn.reshape(1, H))


def _dense_mid(xa, xb, sa, sb, t, degmat, A1, A3, B16, Wu1, Wu2, Wn2,
               cvec, bu, bn2):
    NP, Hh = xa.shape
    H = 2 * Hh
    EF = t.shape[1]

    def body(xa_r, xb_r, sa_r, sb_r, t_r, dg_r, A1_r, A3_r, B16_r, Wu1_r,
             Wu2_r, Wn2_r, cv_r, bu_r, bn2_r, ya_r, yb_r):
        x1 = jnp.concatenate([xa_r[...], xb_r[...]], axis=1)
        s0 = jnp.concatenate([sa_r[...], sb_r[...]], axis=1) + x1
        dg = dg_r[...][:, 0:1] + 1.0
        dot = functools.partial(jnp.dot, preferred_element_type=jnp.float32)
        aggr = (dg * (dot(x1, A1_r[...]) + cv_r[...])
                + dot(t_r[...], B16_r[...]) + dot(s0, A3_r[...]))
        h = jax.nn.relu(dot(x1, Wu1_r[...]) + dot(aggr, Wu2_r[...]) + bu_r[...])
        y = dot(h, Wn2_r[...]) + bn2_r[...]
        ya_r[...] = y[:, :Hh]
        yb_r[...] = y[:, Hh:]

    row = lambda i: (i, 0)
    fix = lambda i: (0, 0)
    return pl.pallas_call(
        body,
        grid=(NP // _R,),
        in_specs=[
            pl.BlockSpec((_R, Hh), row), pl.BlockSpec((_R, Hh), row),
            pl.BlockSpec((_R, Hh), row), pl.BlockSpec((_R, Hh), row),
            pl.BlockSpec((_R, EF), row), pl.BlockSpec((_R, EF), row),
            pl.BlockSpec((H, H), fix), pl.BlockSpec((H, H), fix),
            pl.BlockSpec((EF, H), fix), pl.BlockSpec((H, H), fix),
            pl.BlockSpec((H, H), fix), pl.BlockSpec((H, H), fix),
            pl.BlockSpec((1, H), fix), pl.BlockSpec((1, H), fix),
            pl.BlockSpec((1, H), fix),
        ],
        out_specs=[pl.BlockSpec((_R, Hh), row)] * 2,
        out_shape=[jax.ShapeDtypeStruct((NP, Hh), jnp.float32)] * 2,
    )(xa, xb, sa, sb, t, degmat, A1, A3, B16, Wu1, Wu2, Wn2,
      cvec.reshape(1, H), bu.reshape(1, H), bn2.reshape(1, H))


def _pool(ha, hb, batch_p, n_real):
    NP, Hh = ha.shape
    H = 2 * Hh
    grid = NP // _R

    def body(ha_r, hb_r, b_r, out_ref, cnt_ref):
        i = pl.program_id(0)

        @pl.when(i == 0)
        def _():
            out_ref[...] = jnp.zeros_like(out_ref)
            cnt_ref[...] = jnp.zeros_like(cnt_ref)

        h = jnp.concatenate([ha_r[...], hb_r[...]], axis=1)
        b = b_r[...]
        rows = i * _R + lax.broadcasted_iota(jnp.int32, (_R, 1), 0)
        valid = rows < n_real
        gids = lax.broadcasted_iota(jnp.int32, (_R, _G), 1)
        onehot = jnp.where((b == gids) & valid, 1.0, 0.0)
        dn = (((0,), (0,)), ((), ()))
        out_ref[...] += lax.dot_general(onehot, h, dn,
                                        preferred_element_type=jnp.float32)
        ones_col = jnp.where(valid, 1.0, 0.0)
        cnt_ref[...] += lax.dot_general(onehot, ones_col, dn,
                                        preferred_element_type=jnp.float32)

        @pl.when(i == grid - 1)
        def _():
            out_ref[...] = out_ref[...] / jnp.maximum(cnt_ref[...], 1.0)

    return pl.pallas_call(
        body,
        grid=(grid,),
        in_specs=[
            pl.BlockSpec((_R, Hh), lambda i: (i, 0)),
            pl.BlockSpec((_R, Hh), lambda i: (i, 0)),
            pl.BlockSpec((_R, 1), lambda i: (i, 0)),
        ],
        out_specs=pl.BlockSpec((_G, H), lambda i: (0, 0)),
        out_shape=jax.ShapeDtypeStruct((_G, H), jnp.float32),
        scratch_shapes=[pltpu.VMEM((_G, 1), jnp.float32)],
    )(ha, hb, batch_p)


def _fold(p):
    H = p['Wn'].shape[1]
    Wm1 = p['Wm'][:H]
    Wm2 = p['Wm'][H:2 * H]
    Wm3 = p['Wm'][2 * H:]
    return dict(
        A1=Wm1, A3=Wm3,
        B16=p['We'] @ Wm2,
        cvec=p['bm'] + p['be'] @ Wm2,
        Wu1=p['Wu'][:H], Wu2=p['Wu'][H:],
        bu=p['bu'],
    )


def kernel(x, edge_index, edge_attr, batch, params):
    N, F = x.shape
    E = edge_index.shape[1]
    H = params['c1']['Wn'].shape[1]
    EF = edge_attr.shape[1]
    Hh = H // 2

    NP = _ceil_to(N, _R * _TILES)
    CPT = _ceil_to(E, _TILES * _CW * _K) // (_TILES * _CW)
    EP = _TILES * CPT * _CW
    pe = EP - E

    src = edge_index[0]
    dst = edge_index[1]
    pad_ids = jnp.arange(pe, dtype=jnp.int32)
    src_p = jnp.concatenate([src, pad_ids % 64]).reshape(EP // _CW, _CW)
    dst_p = jnp.concatenate([dst, N + (pad_ids % 8)]).reshape(EP // _CW, _CW)
    ea_p = jnp.concatenate(
        [edge_attr, jnp.zeros((pe, EF), jnp.float32)]).reshape(
            EP // _CW, _CW, EF)

    x_p = jnp.pad(x, ((0, NP - N), (0, 0)))
    batch_p = jnp.pad(batch, (0, NP - N)).reshape(NP, 1)
    zH = jnp.zeros((NP, Hh), jnp.float32)
    zE = jnp.zeros((NP, EF), jnp.float32)
    onesE = jnp.ones((_CW, EF), jnp.float32)

    seg = _sc_seg(NP, CPT, Hh)
    degt = _sc_degt(NP, CPT, EF)

    p1, p2, p3 = params['c1'], params['c2'], params['c3']
    xa, xb = _dense_in(x_p, p1['Wn'], p1['bn'])
    t, degmat = degt(ea_p, dst_p, zE, onesE)

    eye = jnp.eye(H, dtype=jnp.float32)
    zb = jnp.zeros((H,), jnp.float32)
    for p, pn in ((p1, p2), (p2, p3), (p3, None)):
        f = _fold(p)
        sa, sb = seg(xa, xb, src_p, dst_p, zH)
        Wn2 = pn['Wn'] if pn is not None else eye
        bn2 = pn['bn'] if pn is not None else zb
        xa, xb = _dense_mid(xa, xb, sa, sb, t, degmat, f['A1'], f['A3'],
                            f['B16'], f['Wu1'], f['Wu2'], Wn2, f['cvec'],
                            f['bu'], bn2)

    return _pool(xa, xb, batch_p, N)


# speedup vs baseline: 8.0973x; 8.0973x over previous
import functools

import jax
import jax.numpy as jnp
from jax import lax
from jax.experimental import pallas as pl
from jax.experimental.pallas import tpu as pltpu
from jax.experimental.pallas import tpu_sc as plsc

_G = 64
_R = 256
_TILES = 16
_CW = 128
_K = 8
_Q = 4


def _ceil_to(a, b):
    return (a + b - 1) // b * b


def _sc_seg(NP, CPT, W):
    RPT = NP // _TILES
    mesh = plsc.VectorSubcoreMesh(core_axis_name="c", subcore_axis_name="s")

    @functools.partial(
        pl.kernel,
        out_type=[jax.ShapeDtypeStruct((NP, W), jnp.float32)] * _Q,
        mesh=mesh,
        compiler_params=pltpu.CompilerParams(use_tc_tiling_on_sc=False),
        scratch_types=[
            pltpu.VMEM((_K, _CW), jnp.int32),
            pltpu.VMEM((_K, _CW), jnp.int32),
            pltpu.VMEM((_K, _CW, W), jnp.float32),
            pltpu.VMEM_SHARED((NP, W), jnp.float32),
            pltpu.SemaphoreType.DMA,
        ],
    )
    def seg(x0, x1, x2, x3, srcr, dstr, z, s0, s1, s2, s3,
            sbuf, dbuf, rbuf, acc, sem):
        c = lax.axis_index("c")
        s = lax.axis_index("s")
        my = pl.ds(s * RPT, RPT)

        def one_pass(table, out):
            pltpu.sync_copy(z.at[my], acc.at[my])
            plsc.subcore_barrier()

            def body(g, carry):
                row0 = s * CPT + g * _K
                pltpu.sync_copy(srcr.at[pl.ds(row0, _K)], sbuf)
                pltpu.sync_copy(dstr.at[pl.ds(row0, _K)], dbuf)
                cps = [
                    pltpu.async_copy(table.at[sbuf.at[j]], rbuf.at[j], sem)
                    for j in range(_K)
                ]
                for cp in cps:
                    cp.wait()
                for j in range(_K):
                    pltpu.sync_copy(rbuf.at[j], acc.at[dbuf.at[j]], add=True)
                return carry

            lax.fori_loop(0, CPT // _K, body, 0)
            plsc.subcore_barrier()
            pltpu.sync_copy(acc.at[my], out.at[my])

        @pl.when(c == 0)
        def _():
            one_pass(x0, s0)

        @pl.when(c == 1)
        def _():
            one_pass(x1, s1)

        @pl.when(c == 0)
        def _():
            one_pass(x2, s2)

        @pl.when(c == 1)
        def _():
            one_pass(x3, s3)

    return seg


def _sc_degt(NP, CPT, EF):
    RPT = NP // _TILES
    mesh = plsc.VectorSubcoreMesh(core_axis_name="c", subcore_axis_name="s")

    @functools.partial(
        pl.kernel,
        out_type=[jax.ShapeDtypeStruct((NP, EF), jnp.float32)] * 2,
        mesh=mesh,
        compiler_params=pltpu.CompilerParams(use_tc_tiling_on_sc=False),
        scratch_types=[
            pltpu.VMEM((_K, _CW), jnp.int32),
            pltpu.VMEM((_K, _CW, EF), jnp.float32),
            pltpu.VMEM((_CW, EF), jnp.float32),
            pltpu.VMEM_SHARED((NP, EF), jnp.float32),
        ],
    )
    def degt(ear, dstr, z, ones, t_out, deg_out, dbuf, eabuf, obuf, acc):
        c = lax.axis_index("c")
        s = lax.axis_index("s")
        my = pl.ds(s * RPT, RPT)
        pltpu.sync_copy(z.at[my], acc.at[my])

        @pl.when(c == 1)
        def _():
            pltpu.sync_copy(ones, obuf)

        plsc.subcore_barrier()

        def run(use_ea):
            def body(g, carry):
                row0 = s * CPT + g * _K
                pltpu.sync_copy(dstr.at[pl.ds(row0, _K)], dbuf)
                if use_ea:
                    pltpu.sync_copy(ear.at[pl.ds(row0, _K)], eabuf)
                for j in range(_K):
                    src = eabuf.at[j] if use_ea else obuf
                    pltpu.sync_copy(src, acc.at[dbuf.at[j]], add=True)
                return carry

            lax.fori_loop(0, CPT // _K, body, 0)
            plsc.subcore_barrier()

        @pl.when(c == 0)
        def _():
            run(True)
            pltpu.sync_copy(acc.at[my], t_out.at[my])

        @pl.when(c == 1)
        def _():
            run(False)
            pltpu.sync_copy(acc.at[my], deg_out.at[my])

    return degt


def _split_store(y, refs, W):
    for q, r in enumerate(refs):
        r[...] = y[:, q * W:(q + 1) * W]


def _dense_in(x_p, Wn, bn):
    NP, F = x_p.shape
    H = Wn.shape[1]
    W = H // _Q

    def body(x_ref, w_ref, b_ref, *yrefs):
        y = jnp.dot(x_ref[...], w_ref[...],
                    preferred_element_type=jnp.float32) + b_ref[...]
        _split_store(y, yrefs, W)

    return pl.pallas_call(
        body,
        grid=(NP // _R,),
        in_specs=[
            pl.BlockSpec((_R, F), lambda i: (i, 0)),
            pl.BlockSpec((F, H), lambda i: (0, 0)),
            pl.BlockSpec((1, H), lambda i: (0, 0)),
        ],
        out_specs=[pl.BlockSpec((_R, W), lambda i: (i, 0))] * _Q,
        out_shape=[jax.ShapeDtypeStruct((NP, W), jnp.float32)] * _Q,
    )(x_p, Wn, bn.reshape(1, H))


def _dense_mid(xq, sq, t, degmat, A1, A3, B16, Wu1, Wu2, Wn2, cvec, bu, bn2):
    NP, W = xq[0].shape
    H = _Q * W
    EF = t.shape[1]

    def body(x0, x1, x2, x3, s0, s1, s2, s3, t_r, dg_r, A1_r, A3_r, B16_r,
             Wu1_r, Wu2_r, Wn2_r, cv_r, bu_r, bn2_r, *yrefs):
        xcat = jnp.concatenate([x0[...], x1[...], x2[...], x3[...]], axis=1)
        scat = jnp.concatenate([s0[...], s1[...], s2[...], s3[...]],
                               axis=1) + xcat
        dg = dg_r[...][:, 0:1] + 1.0
        dot = functools.partial(jnp.dot, preferred_element_type=jnp.float32)
        aggr = (dg * (dot(xcat, A1_r[...]) + cv_r[...])
                + dot(t_r[...], B16_r[...]) + dot(scat, A3_r[...]))
        h = jax.nn.relu(dot(xcat, Wu1_r[...]) + dot(aggr, Wu2_r[...])
                        + bu_r[...])
        y = dot(h, Wn2_r[...]) + bn2_r[...]
        _split_store(y, yrefs, W)

    row = lambda i: (i, 0)
    fix = lambda i: (0, 0)
    return pl.pallas_call(
        body,
        grid=(NP // _R,),
        in_specs=(
            [pl.BlockSpec((_R, W), row)] * (2 * _Q)
            + [pl.BlockSpec((_R, EF), row)] * 2
            + [pl.BlockSpec((H, H), fix), pl.BlockSpec((H, H), fix),
               pl.BlockSpec((EF, H), fix), pl.BlockSpec((H, H), fix),
               pl.BlockSpec((H, H), fix), pl.BlockSpec((H, H), fix),
               pl.BlockSpec((1, H), fix), pl.BlockSpec((1, H), fix),
               pl.BlockSpec((1, H), fix)]
        ),
        out_specs=[pl.BlockSpec((_R, W), row)] * _Q,
        out_shape=[jax.ShapeDtypeStruct((NP, W), jnp.float32)] * _Q,
    )(*xq, *sq, t, degmat, A1, A3, B16, Wu1, Wu2, Wn2,
      cvec.reshape(1, H), bu.reshape(1, H), bn2.reshape(1, H))


def _pool(hq, batch_p, n_real):
    NP, W = hq[0].shape
    H = _Q * W
    grid = NP // _R

    def body(h0, h1, h2, h3, b_r, out_ref, cnt_ref):
        i = pl.program_id(0)

        @pl.when(i == 0)
        def _():
            out_ref[...] = jnp.zeros_like(out_ref)
            cnt_ref[...] = jnp.zeros_like(cnt_ref)

        h = jnp.concatenate([h0[...], h1[...], h2[...], h3[...]], axis=1)
        b = b_r[...]
        rows = i * _R + lax.broadcasted_iota(jnp.int32, (_R, 1), 0)
        valid = rows < n_real
        gids = lax.broadcasted_iota(jnp.int32, (_R, _G), 1)
        onehot = jnp.where((b == gids) & valid, 1.0, 0.0)
        dn = (((0,), (0,)), ((), ()))
        out_ref[...] += lax.dot_general(onehot, h, dn,
                                        preferred_element_type=jnp.float32)
        ones_col = jnp.where(valid, 1.0, 0.0)
        cnt_ref[...] += lax.dot_general(onehot, ones_col, dn,
                                        preferred_element_type=jnp.float32)

        @pl.when(i == grid - 1)
        def _():
            out_ref[...] = out_ref[...] / jnp.maximum(cnt_ref[...], 1.0)

    return pl.pallas_call(
        body,
        grid=(grid,),
        in_specs=(
            [pl.BlockSpec((_R, W), lambda i: (i, 0))] * _Q
            + [pl.BlockSpec((_R, 1), lambda i: (i, 0))]
        ),
        out_specs=pl.BlockSpec((_G, H), lambda i: (0, 0)),
        out_shape=jax.ShapeDtypeStruct((_G, H), jnp.float32),
        scratch_shapes=[pltpu.VMEM((_G, 1), jnp.float32)],
    )(*hq, batch_p)


def _fold(p):
    H = p['Wn'].shape[1]
    Wm1 = p['Wm'][:H]
    Wm2 = p['Wm'][H:2 * H]
    Wm3 = p['Wm'][2 * H:]
    return dict(
        A1=Wm1, A3=Wm3,
        B16=p['We'] @ Wm2,
        cvec=p['bm'] + p['be'] @ Wm2,
        Wu1=p['Wu'][:H], Wu2=p['Wu'][H:],
        bu=p['bu'],
    )


def kernel(x, edge_index, edge_attr, batch, params):
    N, F = x.shape
    E = edge_index.shape[1]
    H = params['c1']['Wn'].shape[1]
    EF = edge_attr.shape[1]
    W = H // _Q

    NP = _ceil_to(N, _R * _TILES)
    CPT = _ceil_to(E, _TILES * _CW * _K) // (_TILES * _CW)
    EP = _TILES * CPT * _CW
    pe = EP - E

    src = edge_index[0]
    dst = edge_index[1]
    pad_ids = jnp.arange(pe, dtype=jnp.int32)
    src_p = jnp.concatenate([src, pad_ids % 64]).reshape(EP // _CW, _CW)
    dst_p = jnp.concatenate([dst, N + (pad_ids % 8)]).reshape(EP // _CW, _CW)
    ea_p = jnp.concatenate(
        [edge_attr, jnp.zeros((pe, EF), jnp.float32)]).reshape(
            EP // _CW, _CW, EF)

    x_p = jnp.pad(x, ((0, NP - N), (0, 0)))
    batch_p = jnp.pad(batch, (0, NP - N)).reshape(NP, 1)
    zW = jnp.zeros((NP, W), jnp.float32)
    zE = jnp.zeros((NP, EF), jnp.float32)
    onesE = jnp.ones((_CW, EF), jnp.float32)

    seg = _sc_seg(NP, CPT, W)
    degt = _sc_degt(NP, CPT, EF)

    p1, p2, p3 = params['c1'], params['c2'], params['c3']
    xq = _dense_in(x_p, p1['Wn'], p1['bn'])
    t, degmat = degt(ea_p, dst_p, zE, onesE)

    eye = jnp.eye(H, dtype=jnp.float32)
    zb = jnp.zeros((H,), jnp.float32)
    for p, pn in ((p1, p2), (p2, p3), (p3, None)):
        f = _fold(p)
        sq = seg(*xq, src_p, dst_p, zW)
        Wn2 = pn['Wn'] if pn is not None else eye
        bn2 = pn['bn'] if pn is not None else zb
        xq = _dense_mid(xq, sq, t, degmat, f['A1'], f['A3'], f['B16'],
                        f['Wu1'], f['Wu2'], Wn2, f['cvec'], f['bu'], bn2)

    return _pool(xq, batch_p, N)
